# baseline (speedup 1.0000x reference)
import numpy as np
import jax
import jax.numpy as jnp
from jax.experimental import pallas as pl
from jax.experimental.pallas import tpu as pltpu

_STEPS = 1000


def _schedule_tables():
    scale = np.float32(1000.0 / _STEPS)
    beta = np.linspace(scale * np.float32(0.0001), scale * np.float32(0.02),
                       _STEPS, dtype=np.float32)
    alpha = (np.float32(1.0) - beta).astype(np.float32)
    alpha_cum = np.cumprod(alpha, dtype=np.float32)
    sqrt_ac = np.sqrt(alpha_cum).astype(np.float32)
    sqrt_omac = np.sqrt((np.float32(1.0) - alpha_cum)).astype(np.float32)
    return sqrt_ac, sqrt_omac


_SQRT_AC, _SQRT_OMAC = _schedule_tables()


def _scale_add_kernel(ts_ref, sa_ref, so_ref, x_ref, e_ref, o_ref):
    b = pl.program_id(0)
    t = ts_ref[b]
    a = sa_ref[t]
    c = so_ref[t]
    o_ref[...] = a * x_ref[...] + c * e_ref[...]


def kernel(inputs, eps, ts):
    n = inputs.shape[0]
    flat = int(np.prod(inputs.shape[1:]))
    lanes = 128
    rows = flat // lanes
    x = inputs.reshape(n, rows, lanes)
    e = eps.reshape(n, rows, lanes)

    sa = jnp.asarray(_SQRT_AC)
    so = jnp.asarray(_SQRT_OMAC)

    out = pl.pallas_call(
        _scale_add_kernel,
        grid_spec=pltpu.PrefetchScalarGridSpec(
            num_scalar_prefetch=3,
            grid=(n,),
            in_specs=[
                pl.BlockSpec((1, rows, lanes), lambda b, *_: (b, 0, 0)),
                pl.BlockSpec((1, rows, lanes), lambda b, *_: (b, 0, 0)),
            ],
            out_specs=pl.BlockSpec((1, rows, lanes), lambda b, *_: (b, 0, 0)),
        ),
        out_shape=jax.ShapeDtypeStruct((n, rows, lanes), jnp.float32),
    )(ts, sa, so, x, e)
    return out.reshape(inputs.shape)
